# Initial kernel scaffold; baseline (speedup 1.0000x reference)
import jax
import jax.numpy as jnp
from jax.experimental import pallas as pl


def kernel(cls0, reg0, anchors0, cls1, reg1, anchors1, cls2, reg2, anchors2):
    raise NotImplementedError("write your pallas kernel here")


# speedup vs baseline: 2.5066x; 2.5066x over previous
import functools

import jax
import jax.numpy as jnp
import numpy as np
from jax import lax
from jax.experimental import pallas as pl
from jax.experimental.pallas import tpu as pltpu
from jax.experimental.pallas import tpu_sc as plsc

_C = 22
_HWS = (6400, 1600, 400)
_SIZES = tuple(_C * hw for hw in _HWS)
_NB = (0, _SIZES[0], _SIZES[0] + _SIZES[1])
_TOTAL = sum(_SIZES)
_ROWS = 181
_PAD = _ROWS * 1024
_K = 100


def _build_keymap() -> np.ndarray:
    km = np.full(_PAD, 2**30, dtype=np.int32)
    for nb, hw_sz, sz in zip(_NB, _HWS, _SIZES):
        gl = np.arange(sz)
        c = gl // hw_sz
        hw = gl % hw_sz
        km[nb:nb + sz] = nb + hw * _C + c
    return km.reshape(_ROWS, 8, 128)


_KEYMAP = _build_keymap()


def _topk_body(x_ref, km_ref, sc_ref, off_ref, vm_ref, v_scr):
    f32 = jnp.float32
    i32 = jnp.int32
    x = x_ref[:]
    km = km_ref[:]
    s = 1.0 / (1.0 + jnp.exp(-x))
    v_scr[:] = jnp.where(s > 0.05, s, -1.0)
    lane = lax.broadcasted_iota(i32, (1, 128), 1)
    inf_i = jnp.int32(2**30 + 2**29)

    def body(i, carry):
        outv, outk = carry
        v = v_scr[:]
        m = jnp.max(v)
        k = jnp.min(jnp.where(v == m, km, inf_i))
        outv = jnp.where(lane == i, m, outv)
        outk = jnp.where(lane == i, k, outk)
        v_scr[:] = jnp.where(km == k, -2.0, v)
        return outv, outk

    outv, outk = lax.fori_loop(
        0, _K, body,
        (jnp.full((1, 128), -1.0, f32), jnp.zeros((1, 128), i32)))

    validf = (outv > 0.0).astype(f32)
    sc_ref[:] = jnp.sqrt(jnp.maximum(outv * validf, 1e-6))

    lvl = (outk >= _NB[1]).astype(i32) + (outk >= _NB[2]).astype(i32)
    base = jnp.where(lvl == 1, _NB[1], 0) + jnp.where(lvl == 2, _NB[2], 0)
    r = outk - base
    hw = r // _C
    c = r - hw * _C
    hw_sz = (jnp.where(lvl == 0, _HWS[0], 0)
             + jnp.where(lvl == 1, _HWS[1], 0)
             + jnp.where(lvl == 2, _HWS[2], 0))
    regbase = c * 16 * hw_sz + hw
    ab = hw * 4
    rows = [jnp.where(lvl == l, regbase, 0) for l in range(3)]
    rows += [jnp.where(lvl == l, ab, 0) for l in range(3)]
    rows.append((outv > 0.0).astype(i32))
    rows.append(lvl)
    tt = jnp.concatenate(rows, axis=0).T

    iota16 = lax.broadcasted_iota(i32, (128, 16), 1)
    sel8 = (iota16 >= 8).astype(i32)
    for l in range(3):
        off_ref[l] = tt[:, l:l + 1] + iota16 * _HWS[l]
        a1 = tt[:, 3 + l:4 + l] + sel8
        off_ref[3 + l] = a1
        off_ref[6 + l] = a1 + 2
        vm_ref[l] = jnp.broadcast_to(
            (tt[:, 6:7] * (tt[:, 7:8] == l)).astype(f32), (128, 16))


_NC = 2
_NS = 16


def _sc_gather_body(reg0, reg1, reg2, a0, a1, a2, off_hbm, vm_hbm, out_hbm,
                    idx_scr, gat_scr, vm_scr, det_scr, sem):
    wid = lax.axis_index("s") * _NC + lax.axis_index("c")
    base = wid * 4
    for j in range(9):
        pltpu.sync_copy(off_hbm.at[j, pl.ds(wid * 64, 64)], idx_scr.at[j])
    for l in range(3):
        pltpu.sync_copy(vm_hbm.at[l, pl.ds(base, 4)], vm_scr.at[l])
    tables = (reg0, reg1, reg2, a0, a1, a2, a0, a1, a2)
    descs = []
    for j in range(9):
        descs.append(
            pltpu.async_copy(tables[j].at[idx_scr.at[j]], gat_scr.at[j], sem))
    for d in descs:
        d.wait()
    for ci in range(4):
        acc = jnp.zeros((16,), jnp.float32)
        for l in range(3):
            rg = gat_scr[l, pl.ds(ci * 16, 16)]
            p1 = gat_scr[3 + l, pl.ds(ci * 16, 16)]
            p2 = gat_scr[6 + l, pl.ds(ci * 16, 16)]
            vm = vm_scr[l, ci]
            acc = acc + vm * ((p1 + p2) * 0.5 + rg * (p2 - p1))
        det_scr[ci] = acc
    pltpu.sync_copy(det_scr, out_hbm.at[pl.ds(base, 4)])


def kernel(cls0, reg0, anchors0, cls1, reg1, anchors1, cls2, reg2, anchors2):
    f32 = jnp.float32
    xflat = jnp.concatenate([cls0.reshape(-1), cls1.reshape(-1),
                             cls2.reshape(-1)])
    xflat = jnp.pad(xflat, (0, _PAD - _TOTAL), constant_values=-1e30)
    x = xflat.reshape(_ROWS, 8, 128)
    km = jnp.asarray(_KEYMAP)

    sc, offs, vms = pl.pallas_call(
        _topk_body,
        out_shape=[
            jax.ShapeDtypeStruct((1, 128), f32),
            jax.ShapeDtypeStruct((9, 128, 16), jnp.int32),
            jax.ShapeDtypeStruct((3, 128, 16), f32),
        ],
        scratch_shapes=[pltpu.VMEM((_ROWS, 8, 128), f32)],
    )(x, km)

    mesh = plsc.VectorSubcoreMesh(core_axis_name="c", subcore_axis_name="s")
    det = pl.kernel(
        _sc_gather_body,
        out_type=jax.ShapeDtypeStruct((128, 16), f32),
        mesh=mesh,
        scratch_types=[
            pltpu.VMEM((9, 64), jnp.int32),
            pltpu.VMEM((9, 64), f32),
            pltpu.VMEM((3, 4, 16), f32),
            pltpu.VMEM((4, 16), f32),
            pltpu.SemaphoreType.DMA,
        ],
    )(reg0.reshape(-1), reg1.reshape(-1), reg2.reshape(-1),
      anchors0.reshape(-1), anchors1.reshape(-1), anchors2.reshape(-1),
      offs.reshape(9, 2048), vms)

    return jnp.concatenate([det[:_K], sc.reshape(128, 1)[:_K]], axis=1)


# speedup vs baseline: 3.0847x; 1.2306x over previous
import functools

import jax
import jax.numpy as jnp
import numpy as np
from jax import lax
from jax.experimental import pallas as pl
from jax.experimental.pallas import tpu as pltpu
from jax.experimental.pallas import tpu_sc as plsc

_C = 22
_HWS = (6400, 1600, 400)
_SIZES = tuple(_C * hw for hw in _HWS)
_NB = (0, _SIZES[0], _SIZES[0] + _SIZES[1])
_TOTAL = sum(_SIZES)
_ROWS = 181
_PAD = _ROWS * 1024
_K = 100


def _build_keymap() -> np.ndarray:
    km = np.full(_PAD, 2**30, dtype=np.int32)
    for nb, hw_sz, sz in zip(_NB, _HWS, _SIZES):
        gl = np.arange(sz)
        c = gl // hw_sz
        hw = gl % hw_sz
        km[nb:nb + sz] = nb + hw * _C + c
    return km.reshape(_ROWS, 8, 128)


_KEYMAP = _build_keymap()


def _topk_body(x_ref, km_ref, sc_ref, off_ref, vm_ref, v_scr, m_scr, k_scr,
               m2_scr, k2_scr):
    f32 = jnp.float32
    i32 = jnp.int32
    x = x_ref[:]
    km = km_ref[:]
    s = 1.0 / (1.0 + jnp.exp(-x))
    v0 = jnp.where(s > 0.05, s, -1.0)
    v_scr[:] = v0
    lane = lax.broadcasted_iota(i32, (1, 128), 1)
    inf_i = jnp.int32(2**30 + 2**29)
    neg = jnp.float32(-2e30)

    m8 = jnp.max(v0, axis=1)
    k8 = jnp.min(jnp.where(v0 == m8[:, None, :], km, inf_i), axis=1)
    m8 = jnp.concatenate([m8, jnp.full((3, 128), neg)], axis=0)
    k8 = jnp.concatenate([k8, jnp.full((3, 128), inf_i)], axis=0)
    m_scr[:] = m8
    k_scr[:] = k8
    m8b = m8.reshape(23, 8, 128)
    m2 = jnp.max(m8b, axis=1)
    m2_scr[:] = m2
    k2_scr[:] = jnp.min(
        jnp.where(m8b == m2[:, None, :], k8.reshape(23, 8, 128), inf_i), axis=1)

    def body(i, carry):
        outv, outk = carry
        m2v = m2_scr[:]
        m = jnp.max(m2v)
        k = jnp.min(jnp.where(m2v == m, k2_scr[:], inf_i))
        outv = jnp.where(lane == i, m, outv)
        outk = jnp.where(lane == i, k, outk)
        lv = (k >= _NB[1]).astype(i32) + (k >= _NB[2]).astype(i32)
        nb = jnp.where(lv == 1, _NB[1], 0) + jnp.where(lv == 2, _NB[2], 0)
        rr = k - nb
        hw = rr // _C
        c = rr - hw * _C
        hsz = (jnp.where(lv == 0, _HWS[0], 0)
               + jnp.where(lv == 1, _HWS[1], 0)
               + jnp.where(lv == 2, _HWS[2], 0))
        g = nb + c * hsz + hw
        r = g // 1024
        b = r // 8
        row = v_scr[r]
        krow = km_ref[r]
        row2 = jnp.where(krow == k, -2.0, row)
        v_scr[r] = row2
        nm8 = jnp.max(row2, axis=0)
        nk8 = jnp.min(jnp.where(row2 == nm8[None, :], krow, inf_i), axis=0)
        m_scr[r] = nm8
        k_scr[r] = nk8
        blk = m_scr[pl.ds(b * 8, 8)]
        kblk = k_scr[pl.ds(b * 8, 8)]
        nm2 = jnp.max(blk, axis=0)
        nk2 = jnp.min(jnp.where(blk == nm2[None, :], kblk, inf_i), axis=0)
        m2_scr[b] = nm2
        k2_scr[b] = nk2
        return outv, outk

    outv, outk = lax.fori_loop(
        0, _K, body,
        (jnp.full((1, 128), -1.0, f32), jnp.zeros((1, 128), i32)))

    validf = (outv > 0.0).astype(f32)
    sc_ref[:] = jnp.sqrt(jnp.maximum(outv * validf, 1e-6))

    lvl = (outk >= _NB[1]).astype(i32) + (outk >= _NB[2]).astype(i32)
    base = jnp.where(lvl == 1, _NB[1], 0) + jnp.where(lvl == 2, _NB[2], 0)
    r = outk - base
    hw = r // _C
    c = r - hw * _C
    hw_sz = (jnp.where(lvl == 0, _HWS[0], 0)
             + jnp.where(lvl == 1, _HWS[1], 0)
             + jnp.where(lvl == 2, _HWS[2], 0))
    regbase = c * 16 * hw_sz + hw
    ab = hw * 4
    rows = [jnp.where(lvl == l, regbase, 0) for l in range(3)]
    rows += [jnp.where(lvl == l, ab, 0) for l in range(3)]
    rows.append((outv > 0.0).astype(i32))
    rows.append(lvl)
    tt = jnp.concatenate(rows, axis=0).T

    iota16 = lax.broadcasted_iota(i32, (128, 16), 1)
    sel8 = (iota16 >= 8).astype(i32)
    for l in range(3):
        off_ref[l] = tt[:, l:l + 1] + iota16 * _HWS[l]
        a1 = tt[:, 3 + l:4 + l] + sel8
        off_ref[3 + l] = a1
        off_ref[6 + l] = a1 + 2
        vm_ref[l] = jnp.broadcast_to(
            (tt[:, 6:7] * (tt[:, 7:8] == l)).astype(f32), (128, 16))


def _run_topk(x, km, interpret=False):
    f32 = jnp.float32
    return pl.pallas_call(
        _topk_body,
        out_shape=[
            jax.ShapeDtypeStruct((1, 128), f32),
            jax.ShapeDtypeStruct((9, 128, 16), jnp.int32),
            jax.ShapeDtypeStruct((3, 128, 16), f32),
        ],
        scratch_shapes=[
            pltpu.VMEM((_ROWS, 8, 128), f32),
            pltpu.VMEM((184, 128), f32),
            pltpu.VMEM((184, 128), jnp.int32),
            pltpu.VMEM((23, 128), f32),
            pltpu.VMEM((23, 128), jnp.int32),
        ],
        interpret=interpret,
    )(x, km)


_NC = 2
_NS = 16


def _sc_gather_body(reg0, reg1, reg2, a0, a1, a2, off_hbm, vm_hbm, out_hbm,
                    idx_scr, gat_scr, vm_scr, det_scr, sem):
    wid = lax.axis_index("s") * _NC + lax.axis_index("c")
    base = wid * 4
    for j in range(9):
        pltpu.sync_copy(off_hbm.at[j, pl.ds(wid * 64, 64)], idx_scr.at[j])
    for l in range(3):
        pltpu.sync_copy(vm_hbm.at[l, pl.ds(base, 4)], vm_scr.at[l])
    tables = (reg0, reg1, reg2, a0, a1, a2, a0, a1, a2)
    descs = []
    for j in range(9):
        descs.append(
            pltpu.async_copy(tables[j].at[idx_scr.at[j]], gat_scr.at[j], sem))
    for d in descs:
        d.wait()
    for ci in range(4):
        acc = jnp.zeros((16,), jnp.float32)
        for l in range(3):
            rg = gat_scr[l, pl.ds(ci * 16, 16)]
            p1 = gat_scr[3 + l, pl.ds(ci * 16, 16)]
            p2 = gat_scr[6 + l, pl.ds(ci * 16, 16)]
            vm = vm_scr[l, ci]
            acc = acc + vm * ((p1 + p2) * 0.5 + rg * (p2 - p1))
        det_scr[ci] = acc
    pltpu.sync_copy(det_scr, out_hbm.at[pl.ds(base, 4)])


def kernel(cls0, reg0, anchors0, cls1, reg1, anchors1, cls2, reg2, anchors2):
    f32 = jnp.float32
    xflat = jnp.concatenate([cls0.reshape(-1), cls1.reshape(-1),
                             cls2.reshape(-1)])
    xflat = jnp.pad(xflat, (0, _PAD - _TOTAL), constant_values=-1e30)
    x = xflat.reshape(_ROWS, 8, 128)
    km = jnp.asarray(_KEYMAP)

    sc, offs, vms = _run_topk(x, km)

    mesh = plsc.VectorSubcoreMesh(core_axis_name="c", subcore_axis_name="s")
    det = pl.kernel(
        _sc_gather_body,
        out_type=jax.ShapeDtypeStruct((128, 16), f32),
        mesh=mesh,
        scratch_types=[
            pltpu.VMEM((9, 64), jnp.int32),
            pltpu.VMEM((9, 64), f32),
            pltpu.VMEM((3, 4, 16), f32),
            pltpu.VMEM((4, 16), f32),
            pltpu.SemaphoreType.DMA,
        ],
    )(reg0.reshape(-1), reg1.reshape(-1), reg2.reshape(-1),
      anchors0.reshape(-1), anchors1.reshape(-1), anchors2.reshape(-1),
      offs.reshape(9, 2048), vms)

    return jnp.concatenate([det[:_K], sc.reshape(128, 1)[:_K]], axis=1)


# speedup vs baseline: 3.9731x; 1.2880x over previous
import functools

import jax
import jax.numpy as jnp
import numpy as np
from jax import lax
from jax.experimental import pallas as pl
from jax.experimental.pallas import tpu as pltpu
from jax.experimental.pallas import tpu_sc as plsc

_C = 22
_HWS = (6400, 1600, 400)
_SIZES = tuple(_C * hw for hw in _HWS)
_NB = (0, _SIZES[0], _SIZES[0] + _SIZES[1])
_TOTAL = sum(_SIZES)
_ROWS = 181
_PAD = _ROWS * 1024
_K = 100


def _build_keymap() -> np.ndarray:
    km = np.full(_PAD, 2**30, dtype=np.int32)
    for nb, hw_sz, sz in zip(_NB, _HWS, _SIZES):
        gl = np.arange(sz)
        c = gl // hw_sz
        hw = gl % hw_sz
        km[nb:nb + sz] = nb + hw * _C + c
    return km.reshape(_ROWS, 8, 128)


_KEYMAP = _build_keymap()


def _topk_body(x_ref, km_ref, sc_ref, meta_ref, v_scr, m_scr, k_scr,
               m2_scr, k2_scr):
    f32 = jnp.float32
    i32 = jnp.int32
    x = x_ref[:]
    km = km_ref[:]
    s = 1.0 / (1.0 + jnp.exp(-x))
    v0 = jnp.where(s > 0.05, s, -1.0)
    v_scr[:] = v0
    lane = lax.broadcasted_iota(i32, (1, 128), 1)
    inf_i = jnp.int32(2**30 + 2**29)
    neg = jnp.float32(-2e30)

    m8 = jnp.max(v0, axis=1)
    k8 = jnp.min(jnp.where(v0 == m8[:, None, :], km, inf_i), axis=1)
    m8 = jnp.concatenate([m8, jnp.full((3, 128), neg)], axis=0)
    k8 = jnp.concatenate([k8, jnp.full((3, 128), inf_i)], axis=0)
    m_scr[:] = m8
    k_scr[:] = k8
    m8b = m8.reshape(23, 8, 128)
    m2 = jnp.max(m8b, axis=1)
    m2_scr[:] = m2
    k2_scr[:] = jnp.min(
        jnp.where(m8b == m2[:, None, :], k8.reshape(23, 8, 128), inf_i), axis=1)

    def body(i, carry):
        outv, outk = carry
        m2v = m2_scr[:]
        m = jnp.max(m2v)
        k = jnp.min(jnp.where(m2v == m, k2_scr[:], inf_i))
        outv = jnp.where(lane == i, m, outv)
        outk = jnp.where(lane == i, k, outk)
        lv = (k >= _NB[1]).astype(i32) + (k >= _NB[2]).astype(i32)
        nb = jnp.where(lv == 1, _NB[1], 0) + jnp.where(lv == 2, _NB[2], 0)
        rr = k - nb
        hw = rr // _C
        c = rr - hw * _C
        hsz = (jnp.where(lv == 0, _HWS[0], 0)
               + jnp.where(lv == 1, _HWS[1], 0)
               + jnp.where(lv == 2, _HWS[2], 0))
        g = nb + c * hsz + hw
        r = g // 1024
        b = r // 8
        row = v_scr[r]
        krow = km_ref[r]
        row2 = jnp.where(krow == k, -2.0, row)
        v_scr[r] = row2
        nm8 = jnp.max(row2, axis=0)
        nk8 = jnp.min(jnp.where(row2 == nm8[None, :], krow, inf_i), axis=0)
        m_scr[r] = nm8
        k_scr[r] = nk8
        blk = m_scr[pl.ds(b * 8, 8)]
        kblk = k_scr[pl.ds(b * 8, 8)]
        nm2 = jnp.max(blk, axis=0)
        nk2 = jnp.min(jnp.where(blk == nm2[None, :], kblk, inf_i), axis=0)
        m2_scr[b] = nm2
        k2_scr[b] = nk2
        return outv, outk

    outv, outk = lax.fori_loop(
        0, _K, body,
        (jnp.full((1, 128), -1.0, f32), jnp.zeros((1, 128), i32)))

    validf = (outv > 0.0).astype(f32)
    sc_ref[:] = jnp.sqrt(jnp.maximum(outv * validf, 1e-6))

    lvl = (outk >= _NB[1]).astype(i32) + (outk >= _NB[2]).astype(i32)
    base = jnp.where(lvl == 1, _NB[1], 0) + jnp.where(lvl == 2, _NB[2], 0)
    r = outk - base
    hw = r // _C
    c = r - hw * _C
    sv = (jnp.where(lvl == 0, 80, 0) + jnp.where(lvl == 1, 40, 0)
          + jnp.where(lvl == 2, 20, 0))
    h = hw // sv
    w = hw - h * sv
    zero = jnp.zeros((1, 128), i32)
    meta_ref[:] = jnp.concatenate(
        [h, w, c, lvl, (outv > 0.0).astype(i32), zero, zero, zero], axis=0)


def _run_topk(x, km, interpret=False):
    f32 = jnp.float32
    return pl.pallas_call(
        _topk_body,
        out_shape=[
            jax.ShapeDtypeStruct((1, 128), f32),
            jax.ShapeDtypeStruct((8, 128), jnp.int32),
        ],
        scratch_shapes=[
            pltpu.VMEM((_ROWS, 8, 128), f32),
            pltpu.VMEM((184, 128), f32),
            pltpu.VMEM((184, 128), jnp.int32),
            pltpu.VMEM((23, 128), f32),
            pltpu.VMEM((23, 128), jnp.int32),
        ],
        interpret=interpret,
    )(x, km)


_HCHUNK = (16, 40, 20)


def _decode_body(meta_ref, r0_ref, r1_ref, r2_ref, a0_ref, a1_ref, a2_ref,
                 det_ref):
    f32 = jnp.float32
    i32 = jnp.int32
    hv = meta_ref[0:1, :]
    wv = meta_ref[1:2, :]
    cv = meta_ref[2:3, :]
    lv = meta_ref[3:4, :]
    validf = meta_ref[4:5, :].astype(f32)
    prec = lax.Precision.HIGHEST

    r16 = jnp.zeros((16, 128), f32)
    a4 = jnp.zeros((4, 128), f32)
    for l, (s, rref, aref) in enumerate(
            ((80, r0_ref, a0_ref), (40, r1_ref, a1_ref), (20, r2_ref, a2_ref))):
        lm = lv == l
        io_s = lax.broadcasted_iota(i32, (s, 128), 0)
        oh = ((io_s == hv) & lm).astype(f32)
        ow = (io_s == wv).astype(f32)
        hc = _HCHUNK[l]
        x = jnp.zeros((_C * 16, 128), f32)
        for h0 in range(0, s, hc):
            blk = rref[:, h0:h0 + hc, :].reshape(_C * 16 * hc, s)
            z = lax.dot_general(blk, ow, (((1,), (0,)), ((), ())),
                                precision=prec)
            z = z.reshape(_C * 16, hc, 128)
            x = x + jnp.sum(z * oh[h0:h0 + hc][None, :, :], axis=1)
        xr = x.reshape(_C, 16, 128)
        oc = ((lax.broadcasted_iota(i32, (_C, 16, 128), 0) == cv)
              & lm).astype(f32)
        r16 = r16 + jnp.sum(xr * oc, axis=0)
        za = lax.dot_general(aref[:].reshape(4 * s, s), ow,
                             (((1,), (0,)), ((), ())), precision=prec)
        za = za.reshape(4, s, 128)
        a4 = a4 + jnp.sum(za * oh[None, :, :], axis=1)

    x1, y1, x2, y2 = a4[0:1], a4[1:2], a4[2:3], a4[3:4]
    isx = lax.broadcasted_iota(i32, (16, 128), 0) < 8
    center = jnp.where(isx, (x1 + x2) * 0.5, (y1 + y2) * 0.5)
    size = jnp.where(isx, x2 - x1, y2 - y1)
    det_ref[:] = (center + r16 * size) * validf


def kernel(cls0, reg0, anchors0, cls1, reg1, anchors1, cls2, reg2, anchors2):
    f32 = jnp.float32
    xflat = jnp.concatenate([cls0.reshape(-1), cls1.reshape(-1),
                             cls2.reshape(-1)])
    xflat = jnp.pad(xflat, (0, _PAD - _TOTAL), constant_values=-1e30)
    x = xflat.reshape(_ROWS, 8, 128)
    km = jnp.asarray(_KEYMAP)

    sc, meta = _run_topk(x, km)

    det = pl.pallas_call(
        _decode_body,
        out_shape=jax.ShapeDtypeStruct((16, 128), f32),
    )(meta,
      reg0.reshape(_C * 16, 80, 80),
      reg1.reshape(_C * 16, 40, 40),
      reg2.reshape(_C * 16, 20, 20),
      anchors0.T.reshape(4, 80, 80),
      anchors1.T.reshape(4, 40, 40),
      anchors2.T.reshape(4, 20, 20))

    return jnp.concatenate([det.T[:_K], sc.reshape(128, 1)[:_K]], axis=1)
